# baseline (speedup 1.0000x reference)
import functools

import jax
import jax.numpy as jnp
from jax import lax
from jax.experimental import pallas as pl
from jax.experimental.pallas import tpu as pltpu
from jax.experimental.pallas import tpu_sc as plsc

N_NODES = 10000
N_EDGES = 320000
NUM_RBF = 16
NUM_FEATURES = 128
NUM_ATOM_TYPES = 95
TPAD = 128

NC = 2
NS = 16
NW = NC * NS

E_CHUNK = N_EDGES // 2
PER_W = E_CHUNK // NW
BLK = 16000
NBLK_CHUNK = E_CHUNK // BLK


def _prologue_body(embp_ref, w_ref, wrbf_ref, brbf_ref, b_ref,
                   t1_ref, t2_ref, wc_ref, bc_ref):
    embp = embp_ref[...]
    t1_ref[...] = jnp.dot(embp, w_ref[0:NUM_FEATURES, :],
                          preferred_element_type=jnp.float32
                          ).astype(jnp.bfloat16)
    t2_ref[...] = jnp.dot(embp, w_ref[NUM_FEATURES:2 * NUM_FEATURES, :],
                          preferred_element_type=jnp.float32
                          ).astype(jnp.bfloat16)
    w3 = w_ref[2 * NUM_FEATURES:3 * NUM_FEATURES, :]
    wc_ref[...] = jnp.dot(wrbf_ref[...], w3, preferred_element_type=jnp.float32)
    bc_ref[...] = jnp.dot(brbf_ref[...], w3,
                          preferred_element_type=jnp.float32) + b_ref[...]


_prologue = pl.pallas_call(
    _prologue_body,
    out_shape=(
        jax.ShapeDtypeStruct((TPAD, NUM_FEATURES), jnp.bfloat16),
        jax.ShapeDtypeStruct((TPAD, NUM_FEATURES), jnp.bfloat16),
        jax.ShapeDtypeStruct((NUM_RBF, NUM_FEATURES), jnp.float32),
        jax.ShapeDtypeStruct((1, NUM_FEATURES), jnp.float32),
    ),
)


def _sc_gather_body(start, z_hbm, ii_hbm, jj_hbm, zi_hbm, zj_hbm,
                    z_v, ii_v, jj_v, zi_v, zj_v, sem_z, sem_i, sem_j):
    wid = lax.axis_index("s") * NC + lax.axis_index("c")
    base = start + wid * PER_W
    cp_z = pltpu.async_copy(z_hbm, z_v, sem_z)
    cp_i = pltpu.async_copy(ii_hbm.at[pl.ds(base, PER_W)], ii_v, sem_i)
    cp_j = pltpu.async_copy(jj_hbm.at[pl.ds(base, PER_W)], jj_v, sem_j)
    cp_z.wait()
    cp_i.wait()
    cp_j.wait()

    @plsc.parallel_loop(0, PER_W - 16, step=16, unroll=8)
    def _gather_loop(off):
        zi_v[pl.ds(off, 16)] = plsc.load_gather(z_v, [ii_v[pl.ds(off, 16)]])
        zj_v[pl.ds(off, 16)] = plsc.load_gather(z_v, [jj_v[pl.ds(off, 16)]])

    tail = PER_W - 16
    zi_v[pl.ds(tail, 16)] = plsc.load_gather(z_v, [ii_v[pl.ds(tail, 16)]])
    zj_v[pl.ds(tail, 16)] = plsc.load_gather(z_v, [jj_v[pl.ds(tail, 16)]])

    out_base = wid * PER_W
    cpo_i = pltpu.async_copy(zi_v, zi_hbm.at[pl.ds(out_base, PER_W)], sem_i)
    cpo_j = pltpu.async_copy(zj_v, zj_hbm.at[pl.ds(out_base, PER_W)], sem_j)
    cpo_i.wait()
    cpo_j.wait()


def _make_sc_gather(start):
    return pl.kernel(
        functools.partial(_sc_gather_body, start),
        out_type=(
            jax.ShapeDtypeStruct((E_CHUNK,), jnp.int32),
            jax.ShapeDtypeStruct((E_CHUNK,), jnp.int32),
        ),
        mesh=plsc.VectorSubcoreMesh(core_axis_name="c", subcore_axis_name="s"),
        compiler_params=pltpu.CompilerParams(needs_layout_passes=False),
        scratch_types=[
            pltpu.VMEM((N_NODES,), jnp.int32),
            pltpu.VMEM((PER_W,), jnp.int32),
            pltpu.VMEM((PER_W,), jnp.int32),
            pltpu.VMEM((PER_W,), jnp.int32),
            pltpu.VMEM((PER_W,), jnp.int32),
            pltpu.SemaphoreType.DMA,
            pltpu.SemaphoreType.DMA,
            pltpu.SemaphoreType.DMA,
        ],
    )


_sc_gather_a = _make_sc_gather(0)
_sc_gather_b = _make_sc_gather(E_CHUNK)


def _main_compute(zi_ref, zj_ref, rbf_ref, t1_ref, t2_ref, wc_ref, bc_ref,
                  out_ref):
    t_iota = lax.broadcasted_iota(jnp.int32, (TPAD, BLK), 0)
    ohi = (jnp.broadcast_to(zi_ref[0], (TPAD, BLK)) == t_iota
           ).astype(jnp.bfloat16)
    acc = lax.dot_general(ohi, t1_ref[...], (((0,), (0,)), ((), ())),
                          preferred_element_type=jnp.float32)
    ohj = (jnp.broadcast_to(zj_ref[0], (TPAD, BLK)) == t_iota
           ).astype(jnp.bfloat16)
    acc = acc + lax.dot_general(ohj, t2_ref[...], (((0,), (0,)), ((), ())),
                                preferred_element_type=jnp.float32)
    acc = acc + jnp.dot(rbf_ref[...], wc_ref[...],
                        preferred_element_type=jnp.float32)
    out_ref[...] = acc + bc_ref[...]


def _main_body_a(zi_ref, zj_ref, rbf_ref, t1_ref, t2_ref, wc_ref, bc_ref,
                 out_ref):
    _main_compute(zi_ref, zj_ref, rbf_ref, t1_ref, t2_ref, wc_ref, bc_ref,
                  out_ref)


def _main_body_b(zi_ref, zj_ref, rbf_ref, t1_ref, t2_ref, wc_ref, bc_ref,
                 prev_ref, out_ref):
    del prev_ref
    _main_compute(zi_ref, zj_ref, rbf_ref, t1_ref, t2_ref, wc_ref, bc_ref,
                  out_ref)


def _common_in_specs(block_off):
    return [
        pl.BlockSpec((1, 1, BLK), lambda i: (i, 0, 0)),
        pl.BlockSpec((1, 1, BLK), lambda i: (i, 0, 0)),
        pl.BlockSpec((BLK, NUM_RBF), lambda i: (i + block_off, 0)),
        pl.BlockSpec((TPAD, NUM_FEATURES), lambda i: (0, 0)),
        pl.BlockSpec((TPAD, NUM_FEATURES), lambda i: (0, 0)),
        pl.BlockSpec((NUM_RBF, NUM_FEATURES), lambda i: (0, 0)),
        pl.BlockSpec((1, NUM_FEATURES), lambda i: (0, 0)),
    ]


_main_a = pl.pallas_call(
    _main_body_a,
    grid=(NBLK_CHUNK,),
    in_specs=_common_in_specs(0),
    out_specs=pl.BlockSpec((BLK, NUM_FEATURES), lambda i: (i, 0)),
    out_shape=jax.ShapeDtypeStruct((N_EDGES, NUM_FEATURES), jnp.float32),
    compiler_params=pltpu.CompilerParams(fuse_transposed_lhs_in_matmul=True),
)

_main_b = pl.pallas_call(
    _main_body_b,
    grid=(NBLK_CHUNK,),
    in_specs=_common_in_specs(NBLK_CHUNK)
    + [pl.BlockSpec(memory_space=pl.ANY)],
    out_specs=pl.BlockSpec((BLK, NUM_FEATURES),
                           lambda i: (i + NBLK_CHUNK, 0)),
    out_shape=jax.ShapeDtypeStruct((N_EDGES, NUM_FEATURES), jnp.float32),
    input_output_aliases={7: 0},
    compiler_params=pltpu.CompilerParams(fuse_transposed_lhs_in_matmul=True),
)


def kernel(Z, rbf, idnb_i, idnb_j, embeddings, W_rbf, b_rbf, W, b):
    Z = Z.astype(jnp.int32)
    idnb_i = idnb_i.astype(jnp.int32)
    idnb_j = idnb_j.astype(jnp.int32)
    embp = jnp.zeros((TPAD, NUM_FEATURES), jnp.float32
                     ).at[:NUM_ATOM_TYPES].set(embeddings)
    t1, t2, wc, bc = _prologue(embp, W, W_rbf,
                               b_rbf.reshape(1, NUM_FEATURES),
                               b.reshape(1, NUM_FEATURES))
    zi_a, zj_a = _sc_gather_a(Z, idnb_i, idnb_j)
    zi_b, zj_b = _sc_gather_b(Z, idnb_i, idnb_j)
    out = _main_a(zi_a.reshape(NBLK_CHUNK, 1, BLK),
                  zj_a.reshape(NBLK_CHUNK, 1, BLK),
                  rbf, t1, t2, wc, bc)
    out = _main_b(zi_b.reshape(NBLK_CHUNK, 1, BLK),
                  zj_b.reshape(NBLK_CHUNK, 1, BLK),
                  rbf, t1, t2, wc, bc, out)
    return out


# speedup vs baseline: 1.0678x; 1.0678x over previous
import functools

import jax
import jax.numpy as jnp
from jax import lax
from jax.experimental import pallas as pl
from jax.experimental.pallas import tpu as pltpu
from jax.experimental.pallas import tpu_sc as plsc

N_NODES = 10000
N_EDGES = 320000
NUM_RBF = 16
NUM_FEATURES = 128
NUM_ATOM_TYPES = 95
TPAD = 128

NC = 2
NS = 16
NW = NC * NS
EDGES_PER_WORKER = N_EDGES // NW

BLK = 16000
NBLK = N_EDGES // BLK


def _sc_gather_body(z_hbm, ii_hbm, jj_hbm, zi_hbm, zj_hbm,
                    z_v, ii_v, jj_v, zi_v, zj_v, sem_z, sem_i, sem_j):
    wid = lax.axis_index("s") * NC + lax.axis_index("c")
    base = wid * EDGES_PER_WORKER
    cp_z = pltpu.async_copy(z_hbm, z_v, sem_z)
    cp_i = pltpu.async_copy(ii_hbm.at[pl.ds(base, EDGES_PER_WORKER)], ii_v,
                            sem_i)
    cp_j = pltpu.async_copy(jj_hbm.at[pl.ds(base, EDGES_PER_WORKER)], jj_v,
                            sem_j)
    cp_z.wait()
    cp_i.wait()
    cp_j.wait()

    @plsc.parallel_loop(0, EDGES_PER_WORKER, step=16, unroll=8)
    def _gather_loop(off):
        zi_v[pl.ds(off, 16)] = plsc.load_gather(z_v, [ii_v[pl.ds(off, 16)]])
        zj_v[pl.ds(off, 16)] = plsc.load_gather(z_v, [jj_v[pl.ds(off, 16)]])
    cpo_i = pltpu.async_copy(zi_v, zi_hbm.at[pl.ds(base, EDGES_PER_WORKER)],
                             sem_i)
    cpo_j = pltpu.async_copy(zj_v, zj_hbm.at[pl.ds(base, EDGES_PER_WORKER)],
                             sem_j)
    cpo_i.wait()
    cpo_j.wait()


_sc_gather = pl.kernel(
    _sc_gather_body,
    out_type=(
        jax.ShapeDtypeStruct((N_EDGES,), jnp.int32),
        jax.ShapeDtypeStruct((N_EDGES,), jnp.int32),
    ),
    mesh=plsc.VectorSubcoreMesh(core_axis_name="c", subcore_axis_name="s"),
    compiler_params=pltpu.CompilerParams(needs_layout_passes=False),
    scratch_types=[
        pltpu.VMEM((N_NODES,), jnp.int32),
        pltpu.VMEM((EDGES_PER_WORKER,), jnp.int32),
        pltpu.VMEM((EDGES_PER_WORKER,), jnp.int32),
        pltpu.VMEM((EDGES_PER_WORKER,), jnp.int32),
        pltpu.VMEM((EDGES_PER_WORKER,), jnp.int32),
        pltpu.SemaphoreType.DMA,
        pltpu.SemaphoreType.DMA,
        pltpu.SemaphoreType.DMA,
    ],
)


def _main_body(zi_ref, zj_ref, rbf_ref, embp_ref, w_ref, wrbf_ref, brbf_ref,
               b_ref, out_ref, t1_s, t2_s, wc_s, bc_s):
    @pl.when(pl.program_id(0) == 0)
    def _fold_weights():
        embp = embp_ref[...]
        t1_s[...] = jnp.dot(embp, w_ref[0:NUM_FEATURES, :],
                            preferred_element_type=jnp.float32
                            ).astype(jnp.bfloat16)
        t2_s[...] = jnp.dot(embp, w_ref[NUM_FEATURES:2 * NUM_FEATURES, :],
                            preferred_element_type=jnp.float32
                            ).astype(jnp.bfloat16)
        w3 = w_ref[2 * NUM_FEATURES:3 * NUM_FEATURES, :]
        wc_s[...] = jnp.dot(wrbf_ref[...], w3,
                            preferred_element_type=jnp.float32)
        bc_s[...] = jnp.dot(brbf_ref[...], w3,
                            preferred_element_type=jnp.float32) + b_ref[...]

    t_iota = lax.broadcasted_iota(jnp.int32, (TPAD, BLK), 0)
    ohi = (jnp.broadcast_to(zi_ref[0], (TPAD, BLK)) == t_iota
           ).astype(jnp.bfloat16)
    acc = lax.dot_general(ohi, t1_s[...], (((0,), (0,)), ((), ())),
                          preferred_element_type=jnp.float32)
    ohj = (jnp.broadcast_to(zj_ref[0], (TPAD, BLK)) == t_iota
           ).astype(jnp.bfloat16)
    acc = acc + lax.dot_general(ohj, t2_s[...], (((0,), (0,)), ((), ())),
                                preferred_element_type=jnp.float32)
    acc = acc + jnp.dot(rbf_ref[...], wc_s[...],
                        preferred_element_type=jnp.float32)
    out_ref[...] = acc + bc_s[...]


_main = pl.pallas_call(
    _main_body,
    grid=(NBLK,),
    in_specs=[
        pl.BlockSpec((1, 1, BLK), lambda i: (i, 0, 0)),
        pl.BlockSpec((1, 1, BLK), lambda i: (i, 0, 0)),
        pl.BlockSpec((BLK, NUM_RBF), lambda i: (i, 0)),
        pl.BlockSpec((TPAD, NUM_FEATURES), lambda i: (0, 0)),
        pl.BlockSpec((3 * NUM_FEATURES, NUM_FEATURES), lambda i: (0, 0)),
        pl.BlockSpec((NUM_RBF, NUM_FEATURES), lambda i: (0, 0)),
        pl.BlockSpec((1, NUM_FEATURES), lambda i: (0, 0)),
        pl.BlockSpec((1, NUM_FEATURES), lambda i: (0, 0)),
    ],
    out_specs=pl.BlockSpec((BLK, NUM_FEATURES), lambda i: (i, 0)),
    out_shape=jax.ShapeDtypeStruct((N_EDGES, NUM_FEATURES), jnp.float32),
    scratch_shapes=[
        pltpu.VMEM((TPAD, NUM_FEATURES), jnp.bfloat16),
        pltpu.VMEM((TPAD, NUM_FEATURES), jnp.bfloat16),
        pltpu.VMEM((NUM_RBF, NUM_FEATURES), jnp.float32),
        pltpu.VMEM((1, NUM_FEATURES), jnp.float32),
    ],
    compiler_params=pltpu.CompilerParams(fuse_transposed_lhs_in_matmul=True),
)


def kernel(Z, rbf, idnb_i, idnb_j, embeddings, W_rbf, b_rbf, W, b):
    Z = Z.astype(jnp.int32)
    idnb_i = idnb_i.astype(jnp.int32)
    idnb_j = idnb_j.astype(jnp.int32)
    embp = jnp.zeros((TPAD, NUM_FEATURES), jnp.float32
                     ).at[:NUM_ATOM_TYPES].set(embeddings)
    zi, zj = _sc_gather(Z, idnb_i, idnb_j)
    out = _main(zi.reshape(NBLK, 1, BLK), zj.reshape(NBLK, 1, BLK),
                rbf, embp, W, W_rbf,
                b_rbf.reshape(1, NUM_FEATURES),
                b.reshape(1, NUM_FEATURES))
    return out


# speedup vs baseline: 1.0824x; 1.0137x over previous
import functools

import jax
import jax.numpy as jnp
from jax import lax
from jax.experimental import pallas as pl
from jax.experimental.pallas import tpu as pltpu
from jax.experimental.pallas import tpu_sc as plsc

N_NODES = 10000
N_EDGES = 320000
NUM_RBF = 16
NUM_FEATURES = 128
NUM_ATOM_TYPES = 95
TPAD = 128

NC = 2
NS = 16
NW = NC * NS
EDGES_PER_WORKER = N_EDGES // NW

BLK = 16000
NBLK = N_EDGES // BLK


def _sc_gather_body(z_hbm, ii_hbm, jj_hbm, zc_hbm,
                    z_v, ii_v, jj_v, zc_v, sem_z, sem_i, sem_j):
    wid = lax.axis_index("s") * NC + lax.axis_index("c")
    base = wid * EDGES_PER_WORKER
    cp_z = pltpu.async_copy(z_hbm, z_v, sem_z)
    cp_i = pltpu.async_copy(ii_hbm.at[pl.ds(base, EDGES_PER_WORKER)], ii_v,
                            sem_i)
    cp_j = pltpu.async_copy(jj_hbm.at[pl.ds(base, EDGES_PER_WORKER)], jj_v,
                            sem_j)
    cp_z.wait()
    cp_i.wait()
    cp_j.wait()

    @plsc.parallel_loop(0, EDGES_PER_WORKER, step=16, unroll=8)
    def _gather_loop(off):
        zi = plsc.load_gather(z_v, [ii_v[pl.ds(off, 16)]])
        zj = plsc.load_gather(z_v, [jj_v[pl.ds(off, 16)]])
        zc_v[pl.ds(off, 16)] = zi | (zj << 8)
    cpo = pltpu.async_copy(zc_v, zc_hbm.at[pl.ds(base, EDGES_PER_WORKER)],
                           sem_i)
    cpo.wait()


_sc_gather = pl.kernel(
    _sc_gather_body,
    out_type=jax.ShapeDtypeStruct((N_EDGES,), jnp.int32),
    mesh=plsc.VectorSubcoreMesh(core_axis_name="c", subcore_axis_name="s"),
    compiler_params=pltpu.CompilerParams(needs_layout_passes=False),
    scratch_types=[
        pltpu.VMEM((N_NODES,), jnp.int32),
        pltpu.VMEM((EDGES_PER_WORKER,), jnp.int32),
        pltpu.VMEM((EDGES_PER_WORKER,), jnp.int32),
        pltpu.VMEM((EDGES_PER_WORKER,), jnp.int32),
        pltpu.SemaphoreType.DMA,
        pltpu.SemaphoreType.DMA,
        pltpu.SemaphoreType.DMA,
    ],
)


def _main_body(zc_ref, rbf_ref, embp_ref, w_ref, wrbf_ref, brbf_ref,
               b_ref, out_ref, t1_s, t2_s, wc_s, bc_s):
    @pl.when(pl.program_id(0) == 0)
    def _fold_weights():
        embp = embp_ref[...]
        t1_s[...] = jnp.dot(embp, w_ref[0:NUM_FEATURES, :],
                            preferred_element_type=jnp.float32
                            ).astype(jnp.bfloat16)
        t2_s[...] = jnp.dot(embp, w_ref[NUM_FEATURES:2 * NUM_FEATURES, :],
                            preferred_element_type=jnp.float32
                            ).astype(jnp.bfloat16)
        w3 = w_ref[2 * NUM_FEATURES:3 * NUM_FEATURES, :]
        wc_s[...] = jnp.dot(wrbf_ref[...], w3,
                            preferred_element_type=jnp.float32)
        bc_s[...] = jnp.dot(brbf_ref[...], w3,
                            preferred_element_type=jnp.float32) + b_ref[...]

    t_iota = lax.broadcasted_iota(jnp.int32, (TPAD, BLK), 0)
    zc = zc_ref[0]
    ohi = (jnp.broadcast_to(zc & 0xFF, (TPAD, BLK)) == t_iota
           ).astype(jnp.bfloat16)
    acc = lax.dot_general(ohi, t1_s[...], (((0,), (0,)), ((), ())),
                          preferred_element_type=jnp.float32)
    ohj = (jnp.broadcast_to(zc >> 8, (TPAD, BLK)) == t_iota
           ).astype(jnp.bfloat16)
    acc = acc + lax.dot_general(ohj, t2_s[...], (((0,), (0,)), ((), ())),
                                preferred_element_type=jnp.float32)
    acc = acc + jnp.dot(rbf_ref[...], wc_s[...],
                        preferred_element_type=jnp.float32)
    out_ref[...] = acc + bc_s[...]


_main = pl.pallas_call(
    _main_body,
    grid=(NBLK,),
    in_specs=[
        pl.BlockSpec((1, 1, BLK), lambda i: (i, 0, 0)),
        pl.BlockSpec((BLK, NUM_RBF), lambda i: (i, 0)),
        pl.BlockSpec((TPAD, NUM_FEATURES), lambda i: (0, 0)),
        pl.BlockSpec((3 * NUM_FEATURES, NUM_FEATURES), lambda i: (0, 0)),
        pl.BlockSpec((NUM_RBF, NUM_FEATURES), lambda i: (0, 0)),
        pl.BlockSpec((1, NUM_FEATURES), lambda i: (0, 0)),
        pl.BlockSpec((1, NUM_FEATURES), lambda i: (0, 0)),
    ],
    out_specs=pl.BlockSpec((BLK, NUM_FEATURES), lambda i: (i, 0)),
    out_shape=jax.ShapeDtypeStruct((N_EDGES, NUM_FEATURES), jnp.float32),
    scratch_shapes=[
        pltpu.VMEM((TPAD, NUM_FEATURES), jnp.bfloat16),
        pltpu.VMEM((TPAD, NUM_FEATURES), jnp.bfloat16),
        pltpu.VMEM((NUM_RBF, NUM_FEATURES), jnp.float32),
        pltpu.VMEM((1, NUM_FEATURES), jnp.float32),
    ],
    compiler_params=pltpu.CompilerParams(fuse_transposed_lhs_in_matmul=True),
)


def kernel(Z, rbf, idnb_i, idnb_j, embeddings, W_rbf, b_rbf, W, b):
    Z = Z.astype(jnp.int32)
    idnb_i = idnb_i.astype(jnp.int32)
    idnb_j = idnb_j.astype(jnp.int32)
    embp = jnp.zeros((TPAD, NUM_FEATURES), jnp.float32
                     ).at[:NUM_ATOM_TYPES].set(embeddings)
    zc = _sc_gather(Z, idnb_i, idnb_j)
    out = _main(zc.reshape(NBLK, 1, BLK),
                rbf, embp, W, W_rbf,
                b_rbf.reshape(1, NUM_FEATURES),
                b.reshape(1, NUM_FEATURES))
    return out
